# Initial kernel scaffold; baseline (speedup 1.0000x reference)
import jax
import jax.numpy as jnp
from jax.experimental import pallas as pl


def kernel(x, node_time, seed_time, W_enc, b_enc, W_time, b_time, W_self1, W_neigh1, b1, W_self2, W_neigh2, b2, W_mlp1, b_mlp1, ln_g, ln_b, W_mlp2, b_mlp2, batch_ids, edge_index):
    raise NotImplementedError("write your pallas kernel here")


# speedup vs baseline: 2.1864x; 2.1864x over previous
import functools

import jax
import jax.numpy as jnp
import numpy as np
from jax import lax
from jax.experimental import pallas as pl
from jax.experimental.pallas import tpu as pltpu
from jax.experimental.pallas import tpu_sc as plsc

N = 10000
E = 160000
D = 256
NSEED = 1024
H = 128

NC = 2
NS = 16
NW = NC * NS

EPAD = 163840
CE = 80
NROW = 10112
ZROW = NROW // NS
BROW = 1040
ZROWB = BROW // NS

_mesh = plsc.VectorSubcoreMesh(
    core_axis_name="c", subcore_axis_name="s", num_cores=NC, num_subcores=NS)


@functools.partial(
    pl.kernel,
    out_type=(
        jax.ShapeDtypeStruct((NC, NROW, H), jnp.float32),
    ),
    mesh=_mesh,
    scratch_types=[
        pltpu.VMEM_SHARED((NROW, H), jnp.float32),
        pltpu.VMEM((CE, H), jnp.float32),
        pltpu.VMEM((CE,), jnp.int32),
        pltpu.VMEM((CE,), jnp.int32),
        pltpu.SemaphoreType.DMA,
    ],
)
def _agg1_kernel(src_hbm, dst_hbm, t0_hbm, t1_hbm, zrow_hbm,
                 agg_hbm,
                 acc_sh, rows_v, src_v, dst_v, sem):
    c = lax.axis_index("c")
    s = lax.axis_index("s")
    pltpu.sync_copy(zrow_hbm, acc_sh.at[pl.ds(s * ZROW, ZROW)])
    plsc.subcore_barrier()

    nchunk = (EPAD // NS) // CE
    ebase = s * (EPAD // NS)

    def chunk0(k, carry):
        base = ebase + k * CE
        pltpu.sync_copy(src_hbm.at[pl.ds(base, CE)], src_v)
        pltpu.sync_copy(dst_hbm.at[pl.ds(base, CE)], dst_v)
        pltpu.async_copy(t0_hbm.at[src_v], rows_v, sem).wait()
        pltpu.sync_copy(rows_v, acc_sh.at[dst_v], add=True)
        return carry

    def chunk1(k, carry):
        base = ebase + k * CE
        pltpu.sync_copy(src_hbm.at[pl.ds(base, CE)], src_v)
        pltpu.sync_copy(dst_hbm.at[pl.ds(base, CE)], dst_v)
        pltpu.async_copy(t1_hbm.at[src_v], rows_v, sem).wait()
        pltpu.sync_copy(rows_v, acc_sh.at[dst_v], add=True)
        return carry

    @pl.when(c == 0)
    def _():
        lax.fori_loop(0, nchunk, chunk0, 0)

    @pl.when(c == 1)
    def _():
        lax.fori_loop(0, nchunk, chunk1, 0)

    plsc.subcore_barrier()
    pltpu.sync_copy(acc_sh.at[pl.ds(s * ZROW, ZROW)],
                    agg_hbm.at[c, pl.ds(s * ZROW, ZROW)])


@functools.partial(
    pl.kernel,
    out_type=jax.ShapeDtypeStruct((NC, NROW, H), jnp.float32),
    mesh=_mesh,
    scratch_types=[
        pltpu.VMEM_SHARED((NROW, H), jnp.float32),
        pltpu.VMEM((CE, H), jnp.float32),
        pltpu.VMEM((CE,), jnp.int32),
    ],
)
def _deg_kernel(dst_hbm, zrow_hbm, ones_hbm, deg_hbm, acc_sh, ones_v, dst_v):
    c = lax.axis_index("c")
    s = lax.axis_index("s")
    pltpu.sync_copy(zrow_hbm, acc_sh.at[pl.ds(s * ZROW, ZROW)])
    pltpu.sync_copy(ones_hbm, ones_v)
    plsc.subcore_barrier()

    per_w = EPAD // NW
    nchunk = per_w // CE
    ebase = (s * NC + c) * per_w

    def chunk(k, carry):
        base = ebase + k * CE
        pltpu.sync_copy(dst_hbm.at[pl.ds(base, CE)], dst_v)
        pltpu.sync_copy(ones_v, acc_sh.at[dst_v], add=True)
        return carry

    lax.fori_loop(0, nchunk, chunk, 0)
    plsc.subcore_barrier()
    pltpu.sync_copy(acc_sh.at[pl.ds(s * ZROW, ZROW)],
                    deg_hbm.at[c, pl.ds(s * ZROW, ZROW)])


@functools.partial(
    pl.kernel,
    out_type=(
        jax.ShapeDtypeStruct((NC, NSEED, H), jnp.float32),
        jax.ShapeDtypeStruct((NC, NSEED, H), jnp.float32),
    ),
    mesh=_mesh,
    scratch_types=[
        pltpu.VMEM_SHARED((BROW, H), jnp.float32),
        pltpu.VMEM_SHARED((BROW, H), jnp.float32),
        pltpu.VMEM((CE, H), jnp.float32),
        pltpu.VMEM((CE, H), jnp.float32),
        pltpu.VMEM((CE,), jnp.int32),
        pltpu.VMEM((CE,), jnp.int32),
        pltpu.SemaphoreType.DMA,
    ],
)
def _agg2_kernel(src_hbm, dst_hbm, ta_hbm, tb_hbm, zrow_hbm,
                 pa_hbm, pb_hbm,
                 acca_sh, accb_sh, rowsa_v, rowsb_v, src_v, dst_v, sem):
    c = lax.axis_index("c")
    s = lax.axis_index("s")
    pltpu.sync_copy(zrow_hbm, acca_sh.at[pl.ds(s * ZROWB, ZROWB)])
    pltpu.sync_copy(zrow_hbm, accb_sh.at[pl.ds(s * ZROWB, ZROWB)])
    plsc.subcore_barrier()

    per_w = EPAD // NW
    nchunk = per_w // CE
    ebase = (s * NC + c) * per_w

    def chunk(k, carry):
        base = ebase + k * CE
        pltpu.sync_copy(src_hbm.at[pl.ds(base, CE)], src_v)
        pltpu.sync_copy(dst_hbm.at[pl.ds(base, CE)], dst_v)
        for g in range(CE // 16):
            dv = dst_v[pl.ds(g * 16, 16)]
            dst_v[pl.ds(g * 16, 16)] = jnp.where(dv < NSEED, dv, NSEED)
        pltpu.async_copy(ta_hbm.at[src_v], rowsa_v, sem).wait()
        pltpu.async_copy(tb_hbm.at[src_v], rowsb_v, sem).wait()
        pltpu.sync_copy(rowsa_v, acca_sh.at[dst_v], add=True)
        pltpu.sync_copy(rowsb_v, accb_sh.at[dst_v], add=True)
        return carry

    lax.fori_loop(0, nchunk, chunk, 0)
    plsc.subcore_barrier()
    nout = NSEED // NS
    pltpu.sync_copy(acca_sh.at[pl.ds(s * nout, nout)],
                    pa_hbm.at[c, pl.ds(s * nout, nout)])
    pltpu.sync_copy(accb_sh.at[pl.ds(s * nout, nout)],
                    pb_hbm.at[c, pl.ds(s * nout, nout)])


def _tc1_body(x_ref, ids_ref, nt_ref, seed_ref, we_ref, wt_ref, wn_ref,
              ws_ref, b01_ref, b1_ref, t0_ref, t1_ref, hs_ref):
    x = x_ref[...]
    ids = ids_ref[...]
    iota = lax.broadcasted_iota(jnp.int32, (1, NSEED), 1)
    onehot = (ids == iota).astype(jnp.float32)
    st = jnp.dot(onehot, seed_ref[...], preferred_element_type=jnp.float32)
    rel = st - nt_ref[...]
    k = lax.broadcasted_iota(jnp.int32, (1, D // 2), 1).astype(jnp.float32)
    freqs = jnp.exp((-np.log(10000.0) / (D // 2)) * k)
    ang = rel * freqs
    pe = jnp.concatenate([jnp.sin(ang), jnp.cos(ang)], axis=1)
    h0 = (jnp.dot(x, we_ref[...], preferred_element_type=jnp.float32)
          + jnp.dot(pe, wt_ref[...], preferred_element_type=jnp.float32)
          + b01_ref[...])
    hn = jnp.dot(h0, wn_ref[...], preferred_element_type=jnp.float32)
    t0_ref[...] = hn[:, :H]
    t1_ref[...] = hn[:, H:]
    hs_ref[...] = (jnp.dot(h0, ws_ref[...], preferred_element_type=jnp.float32)
                   + b1_ref[...])


def _tc1(x, ids2d, nt2d, seed2d, W_enc, W_time, W_neigh1, W_self1, b01, b1):
    R = 1000
    grid = (N // R,)
    return pl.pallas_call(
        _tc1_body,
        grid=grid,
        in_specs=[
            pl.BlockSpec((R, D), lambda i: (i, 0)),
            pl.BlockSpec((R, 1), lambda i: (i, 0)),
            pl.BlockSpec((R, 1), lambda i: (i, 0)),
            pl.BlockSpec((NSEED, 1), lambda i: (0, 0)),
            pl.BlockSpec((D, D), lambda i: (0, 0)),
            pl.BlockSpec((D, D), lambda i: (0, 0)),
            pl.BlockSpec((D, D), lambda i: (0, 0)),
            pl.BlockSpec((D, D), lambda i: (0, 0)),
            pl.BlockSpec((1, D), lambda i: (0, 0)),
            pl.BlockSpec((1, D), lambda i: (0, 0)),
        ],
        out_specs=[
            pl.BlockSpec((R, H), lambda i: (i, 0)),
            pl.BlockSpec((R, H), lambda i: (i, 0)),
            pl.BlockSpec((R, D), lambda i: (i, 0)),
        ],
        out_shape=[
            jax.ShapeDtypeStruct((N, H), jnp.float32),
            jax.ShapeDtypeStruct((N, H), jnp.float32),
            jax.ShapeDtypeStruct((N, D), jnp.float32),
        ],
        compiler_params=pltpu.CompilerParams(
            dimension_semantics=("parallel",)),
    )(x, ids2d, nt2d, seed2d, W_enc, W_time, W_neigh1, W_self1, b01, b1)


def _tc2_body(hs_ref, a0_ref, a1_ref, d0_ref, d1_ref, wn2_ref,
              h1_ref, na_ref, nb_ref):
    deg = jnp.maximum(d0_ref[...] + d1_ref[...], 1.0)
    agg = jnp.concatenate([a0_ref[...], a1_ref[...]], axis=1) / deg
    h1 = jnp.maximum(hs_ref[...] + agg, 0.0)
    h1_ref[...] = h1
    hn2 = jnp.dot(h1, wn2_ref[...], preferred_element_type=jnp.float32)
    na_ref[...] = hn2[:, :H]
    nb_ref[...] = hn2[:, H:]


def _tc2(hs1b, a0, a1, d0, d1, W_neigh2):
    R = 1000
    grid = (N // R,)
    return pl.pallas_call(
        _tc2_body,
        grid=grid,
        in_specs=[
            pl.BlockSpec((R, D), lambda i: (i, 0)),
            pl.BlockSpec((R, H), lambda i: (i, 0)),
            pl.BlockSpec((R, H), lambda i: (i, 0)),
            pl.BlockSpec((R, 1), lambda i: (i, 0)),
            pl.BlockSpec((R, 1), lambda i: (i, 0)),
            pl.BlockSpec((D, D), lambda i: (0, 0)),
        ],
        out_specs=[
            pl.BlockSpec((R, D), lambda i: (i, 0)),
            pl.BlockSpec((R, H), lambda i: (i, 0)),
            pl.BlockSpec((R, H), lambda i: (i, 0)),
        ],
        out_shape=[
            jax.ShapeDtypeStruct((N, D), jnp.float32),
            jax.ShapeDtypeStruct((N, H), jnp.float32),
            jax.ShapeDtypeStruct((N, H), jnp.float32),
        ],
        compiler_params=pltpu.CompilerParams(
            dimension_semantics=("parallel",)),
    )(hs1b, a0, a1, d0, d1, W_neigh2)


def _tc3_body(h1s_ref, pa0_ref, pa1_ref, pb0_ref, pb1_ref, d0_ref, d1_ref,
              ws2_ref, b2_ref, wm1_ref, bm1_ref, g_ref, bln_ref, wm2_ref,
              bm2_ref, out_ref):
    deg = jnp.maximum(d0_ref[...] + d1_ref[...], 1.0)
    agg = jnp.concatenate(
        [pa0_ref[...] + pa1_ref[...], pb0_ref[...] + pb1_ref[...]], axis=1) / deg
    h2 = jnp.maximum(
        jnp.dot(h1s_ref[...], ws2_ref[...], preferred_element_type=jnp.float32)
        + b2_ref[...] + agg, 0.0)
    z = (jnp.dot(h2, wm1_ref[...], preferred_element_type=jnp.float32)
         + bm1_ref[...])
    mu = jnp.mean(z, axis=1, keepdims=True)
    var = jnp.mean((z - mu) * (z - mu), axis=1, keepdims=True)
    z = (z - mu) * lax.rsqrt(var + 1e-5) * g_ref[...] + bln_ref[...]
    z = jnp.maximum(z, 0.0)
    out_ref[...] = (jnp.dot(z, wm2_ref[...], preferred_element_type=jnp.float32)
                    + bm2_ref[...])


def _tc3(h1s, pa0, pa1, pb0, pb1, d0_s, d1_s, W_self2, b2, W_mlp1, b_mlp1,
         ln_g, ln_b, W_mlp2, b_mlp2):
    return pl.pallas_call(
        _tc3_body,
        out_shape=jax.ShapeDtypeStruct((NSEED, 1), jnp.float32),
    )(h1s, pa0, pa1, pb0, pb1, d0_s, d1_s, W_self2, b2, W_mlp1, b_mlp1, ln_g,
      ln_b, W_mlp2, b_mlp2)


@jax.jit
def kernel(x, node_time, seed_time, W_enc, b_enc, W_time, b_time, W_self1,
           W_neigh1, b1, W_self2, W_neigh2, b2, W_mlp1, b_mlp1, ln_g, ln_b,
           W_mlp2, b_mlp2, batch_ids, edge_index):
    src = edge_index[0].astype(jnp.int32)
    dst = edge_index[1].astype(jnp.int32)

    b01 = (b_enc + b_time).reshape(1, D)
    t0, t1, hs1b = _tc1(x, batch_ids.astype(jnp.int32).reshape(N, 1),
                        node_time.reshape(N, 1), seed_time.reshape(NSEED, 1),
                        W_enc, W_time, W_neigh1, W_self1,
                        b01, b1.reshape(1, D))

    src_p = jnp.concatenate([src, jnp.zeros((EPAD - E,), jnp.int32)])
    dst_p = jnp.concatenate([dst, jnp.full((EPAD - E,), N, jnp.int32)])
    zrow = jnp.zeros((ZROW, H), jnp.float32)
    onesr = jnp.ones((CE, H), jnp.float32)

    degp = _deg_kernel(dst_p, zrow, onesr)
    (agg,) = _agg1_kernel(src_p, dst_p, t0, t1, zrow)

    h1, na, nb = _tc2(hs1b, agg[0, :N], agg[1, :N],
                      degp[0, :N, 0:1], degp[1, :N, 0:1], W_neigh2)

    zrowb = jnp.zeros((ZROWB, H), jnp.float32)
    pa, pb = _agg2_kernel(src_p, dst_p, na, nb, zrowb)

    out = _tc3(h1[:NSEED], pa[0], pa[1], pb[0], pb[1],
               degp[0, :NSEED, 0:1], degp[1, :NSEED, 0:1],
               W_self2, b2.reshape(1, D), W_mlp1, b_mlp1.reshape(1, H),
               ln_g.reshape(1, H), ln_b.reshape(1, H), W_mlp2,
               b_mlp2.reshape(1, 1))
    return out.reshape(NSEED)


# speedup vs baseline: 3.4475x; 1.5768x over previous
import functools

import jax
import jax.numpy as jnp
import numpy as np
from jax import lax
from jax.experimental import pallas as pl
from jax.experimental.pallas import tpu as pltpu
from jax.experimental.pallas import tpu_sc as plsc

N = 10000
E = 160000
D = 256
NSEED = 1024
H = 128

NC = 2
NS = 16
NW = NC * NS

EPAD = 163840
CE = 128
NROW = 10112
ZROW = NROW // NS
BROW = 1040
ZROWB = BROW // NS

_mesh = plsc.VectorSubcoreMesh(
    core_axis_name="c", subcore_axis_name="s", num_cores=NC, num_subcores=NS)


@functools.partial(
    pl.kernel,
    out_type=(
        jax.ShapeDtypeStruct((NC, NROW, H), jnp.float32),
    ),
    mesh=_mesh,
    scratch_types=[
        pltpu.VMEM_SHARED((NROW, H), jnp.float32),
        pltpu.VMEM((CE, H), jnp.float32),
        pltpu.VMEM((CE, H), jnp.float32),
        pltpu.VMEM((CE,), jnp.int32),
        pltpu.VMEM((CE,), jnp.int32),
        pltpu.VMEM((CE,), jnp.int32),
        pltpu.VMEM((CE,), jnp.int32),
        pltpu.SemaphoreType.DMA,
        pltpu.SemaphoreType.DMA,
    ],
)
def _agg1_kernel(src_hbm, dst_hbm, t0_hbm, t1_hbm, zrow_hbm,
                 agg_hbm,
                 acc_sh, rows0_v, rows1_v, src0_v, src1_v, dst0_v, dst1_v,
                 sem0, sem1):
    c = lax.axis_index("c")
    s = lax.axis_index("s")
    pltpu.sync_copy(zrow_hbm, acc_sh.at[pl.ds(s * ZROW, ZROW)])
    plsc.subcore_barrier()

    nchunk = (EPAD // NS) // CE
    ebase = s * (EPAD // NS)

    def run(table_hbm):
        pltpu.sync_copy(src_hbm.at[pl.ds(ebase, CE)], src0_v)
        pltpu.sync_copy(dst_hbm.at[pl.ds(ebase, CE)], dst0_v)
        pltpu.async_copy(table_hbm.at[src0_v], rows0_v, sem0)

        def body(j, carry):
            b1 = ebase + (2 * j + 1) * CE
            pltpu.sync_copy(src_hbm.at[pl.ds(b1, CE)], src1_v)
            pltpu.sync_copy(dst_hbm.at[pl.ds(b1, CE)], dst1_v)
            pltpu.async_copy(table_hbm.at[src1_v], rows1_v, sem1)
            pltpu.make_async_copy(table_hbm.at[src0_v], rows0_v, sem0).wait()
            pltpu.sync_copy(rows0_v, acc_sh.at[dst0_v], add=True)

            @pl.when(j < nchunk // 2 - 1)
            def _():
                b2 = ebase + (2 * j + 2) * CE
                pltpu.sync_copy(src_hbm.at[pl.ds(b2, CE)], src0_v)
                pltpu.sync_copy(dst_hbm.at[pl.ds(b2, CE)], dst0_v)
                pltpu.async_copy(table_hbm.at[src0_v], rows0_v, sem0)

            pltpu.make_async_copy(table_hbm.at[src1_v], rows1_v, sem1).wait()
            pltpu.sync_copy(rows1_v, acc_sh.at[dst1_v], add=True)
            return carry

        lax.fori_loop(0, nchunk // 2, body, 0)

    @pl.when(c == 0)
    def _():
        run(t0_hbm)

    @pl.when(c == 1)
    def _():
        run(t1_hbm)

    plsc.subcore_barrier()
    pltpu.sync_copy(acc_sh.at[pl.ds(s * ZROW, ZROW)],
                    agg_hbm.at[c, pl.ds(s * ZROW, ZROW)])


@functools.partial(
    pl.kernel,
    out_type=jax.ShapeDtypeStruct((NC, NROW, H), jnp.float32),
    mesh=_mesh,
    scratch_types=[
        pltpu.VMEM_SHARED((NROW, H), jnp.float32),
        pltpu.VMEM((CE, H), jnp.float32),
        pltpu.VMEM((CE,), jnp.int32),
        pltpu.VMEM((CE,), jnp.int32),
        pltpu.SemaphoreType.DMA,
        pltpu.SemaphoreType.DMA,
    ],
)
def _deg_kernel(dst_hbm, zrow_hbm, ones_hbm, deg_hbm, acc_sh, ones_v,
                dst0_v, dst1_v, sem0, sem1):
    c = lax.axis_index("c")
    s = lax.axis_index("s")
    pltpu.sync_copy(zrow_hbm, acc_sh.at[pl.ds(s * ZROW, ZROW)])
    pltpu.sync_copy(ones_hbm, ones_v)
    plsc.subcore_barrier()

    per_w = EPAD // NW
    nchunk = per_w // CE
    ebase = (s * NC + c) * per_w

    pltpu.async_copy(dst_hbm.at[pl.ds(ebase, CE)], dst0_v, sem0)

    def body(j, carry):
        k0 = 2 * j
        pltpu.async_copy(dst_hbm.at[pl.ds(ebase + (k0 + 1) * CE, CE)],
                         dst1_v, sem1)
        pltpu.make_async_copy(dst_hbm.at[pl.ds(ebase, CE)], dst0_v,
                              sem0).wait()
        pltpu.sync_copy(ones_v, acc_sh.at[dst0_v], add=True)

        @pl.when(j < nchunk // 2 - 1)
        def _():
            pltpu.async_copy(dst_hbm.at[pl.ds(ebase + (k0 + 2) * CE, CE)],
                             dst0_v, sem0)

        pltpu.make_async_copy(dst_hbm.at[pl.ds(ebase, CE)], dst1_v,
                              sem1).wait()
        pltpu.sync_copy(ones_v, acc_sh.at[dst1_v], add=True)
        return carry

    lax.fori_loop(0, nchunk // 2, body, 0)
    plsc.subcore_barrier()
    pltpu.sync_copy(acc_sh.at[pl.ds(s * ZROW, ZROW)],
                    deg_hbm.at[c, pl.ds(s * ZROW, ZROW)])


@functools.partial(
    pl.kernel,
    out_type=(
        jax.ShapeDtypeStruct((NC, NSEED, H), jnp.float32),
        jax.ShapeDtypeStruct((NC, NSEED, H), jnp.float32),
    ),
    mesh=_mesh,
    scratch_types=[
        pltpu.VMEM_SHARED((BROW, H), jnp.float32),
        pltpu.VMEM_SHARED((BROW, H), jnp.float32),
        pltpu.VMEM((CE, H), jnp.float32),
        pltpu.VMEM((CE, H), jnp.float32),
        pltpu.VMEM((CE, H), jnp.float32),
        pltpu.VMEM((CE, H), jnp.float32),
        pltpu.VMEM((CE,), jnp.int32),
        pltpu.VMEM((CE,), jnp.int32),
        pltpu.VMEM((CE,), jnp.int32),
        pltpu.VMEM((CE,), jnp.int32),
        pltpu.SemaphoreType.DMA,
        pltpu.SemaphoreType.DMA,
    ],
)
def _agg2_kernel(src_hbm, dst_hbm, ta_hbm, tb_hbm, zrow_hbm,
                 pa_hbm, pb_hbm,
                 acca_sh, accb_sh, rowsa0_v, rowsb0_v, rowsa1_v, rowsb1_v,
                 src0_v, src1_v, dst0_v, dst1_v, sem0, sem1):
    c = lax.axis_index("c")
    s = lax.axis_index("s")
    pltpu.sync_copy(zrow_hbm, acca_sh.at[pl.ds(s * ZROWB, ZROWB)])
    pltpu.sync_copy(zrow_hbm, accb_sh.at[pl.ds(s * ZROWB, ZROWB)])
    plsc.subcore_barrier()

    per_w = EPAD // NW
    nchunk = per_w // CE
    ebase = (s * NC + c) * per_w

    def load_idx(k, src_v, dst_v):
        base = ebase + k * CE
        pltpu.sync_copy(src_hbm.at[pl.ds(base, CE)], src_v)
        pltpu.sync_copy(dst_hbm.at[pl.ds(base, CE)], dst_v)
        for g in range(CE // 16):
            dv = dst_v[pl.ds(g * 16, 16)]
            dst_v[pl.ds(g * 16, 16)] = jnp.where(dv < NSEED, dv, NSEED)

    load_idx(0, src0_v, dst0_v)
    pltpu.async_copy(ta_hbm.at[src0_v], rowsa0_v, sem0)
    pltpu.async_copy(tb_hbm.at[src0_v], rowsb0_v, sem0)

    def body(j, carry):
        load_idx(2 * j + 1, src1_v, dst1_v)
        pltpu.async_copy(ta_hbm.at[src1_v], rowsa1_v, sem1)
        pltpu.async_copy(tb_hbm.at[src1_v], rowsb1_v, sem1)
        pltpu.make_async_copy(ta_hbm.at[src0_v], rowsa0_v, sem0).wait()
        pltpu.make_async_copy(tb_hbm.at[src0_v], rowsb0_v, sem0).wait()
        pltpu.sync_copy(rowsa0_v, acca_sh.at[dst0_v], add=True)
        pltpu.sync_copy(rowsb0_v, accb_sh.at[dst0_v], add=True)

        @pl.when(j < nchunk // 2 - 1)
        def _():
            load_idx(2 * j + 2, src0_v, dst0_v)
            pltpu.async_copy(ta_hbm.at[src0_v], rowsa0_v, sem0)
            pltpu.async_copy(tb_hbm.at[src0_v], rowsb0_v, sem0)

        pltpu.make_async_copy(ta_hbm.at[src1_v], rowsa1_v, sem1).wait()
        pltpu.make_async_copy(tb_hbm.at[src1_v], rowsb1_v, sem1).wait()
        pltpu.sync_copy(rowsa1_v, acca_sh.at[dst1_v], add=True)
        pltpu.sync_copy(rowsb1_v, accb_sh.at[dst1_v], add=True)
        return carry

    lax.fori_loop(0, nchunk // 2, body, 0)
    plsc.subcore_barrier()
    nout = NSEED // NS
    pltpu.sync_copy(acca_sh.at[pl.ds(s * nout, nout)],
                    pa_hbm.at[c, pl.ds(s * nout, nout)])
    pltpu.sync_copy(accb_sh.at[pl.ds(s * nout, nout)],
                    pb_hbm.at[c, pl.ds(s * nout, nout)])


def _tc1_body(x_ref, ids_ref, nt_ref, seed_ref, we_ref, wt_ref, wn_ref,
              ws_ref, b01_ref, b1_ref, t0_ref, t1_ref, hs_ref):
    x = x_ref[...]
    ids = ids_ref[...]
    iota = lax.broadcasted_iota(jnp.int32, (1, NSEED), 1)
    onehot = (ids == iota).astype(jnp.float32)
    st = jnp.dot(onehot, seed_ref[...], preferred_element_type=jnp.float32)
    rel = st - nt_ref[...]
    k = lax.broadcasted_iota(jnp.int32, (1, D // 2), 1).astype(jnp.float32)
    freqs = jnp.exp((-np.log(10000.0) / (D // 2)) * k)
    ang = rel * freqs
    pe = jnp.concatenate([jnp.sin(ang), jnp.cos(ang)], axis=1)
    h0 = (jnp.dot(x, we_ref[...], preferred_element_type=jnp.float32)
          + jnp.dot(pe, wt_ref[...], preferred_element_type=jnp.float32)
          + b01_ref[...])
    hn = jnp.dot(h0, wn_ref[...], preferred_element_type=jnp.float32)
    t0_ref[...] = hn[:, :H]
    t1_ref[...] = hn[:, H:]
    hs_ref[...] = (jnp.dot(h0, ws_ref[...], preferred_element_type=jnp.float32)
                   + b1_ref[...])


def _tc1(x, ids2d, nt2d, seed2d, W_enc, W_time, W_neigh1, W_self1, b01, b1):
    R = 1000
    grid = (N // R,)
    return pl.pallas_call(
        _tc1_body,
        grid=grid,
        in_specs=[
            pl.BlockSpec((R, D), lambda i: (i, 0)),
            pl.BlockSpec((R, 1), lambda i: (i, 0)),
            pl.BlockSpec((R, 1), lambda i: (i, 0)),
            pl.BlockSpec((NSEED, 1), lambda i: (0, 0)),
            pl.BlockSpec((D, D), lambda i: (0, 0)),
            pl.BlockSpec((D, D), lambda i: (0, 0)),
            pl.BlockSpec((D, D), lambda i: (0, 0)),
            pl.BlockSpec((D, D), lambda i: (0, 0)),
            pl.BlockSpec((1, D), lambda i: (0, 0)),
            pl.BlockSpec((1, D), lambda i: (0, 0)),
        ],
        out_specs=[
            pl.BlockSpec((R, H), lambda i: (i, 0)),
            pl.BlockSpec((R, H), lambda i: (i, 0)),
            pl.BlockSpec((R, D), lambda i: (i, 0)),
        ],
        out_shape=[
            jax.ShapeDtypeStruct((N, H), jnp.float32),
            jax.ShapeDtypeStruct((N, H), jnp.float32),
            jax.ShapeDtypeStruct((N, D), jnp.float32),
        ],
        compiler_params=pltpu.CompilerParams(
            dimension_semantics=("parallel",)),
    )(x, ids2d, nt2d, seed2d, W_enc, W_time, W_neigh1, W_self1, b01, b1)


def _tc2_body(hs_ref, a0_ref, a1_ref, d0_ref, d1_ref, wn2_ref,
              h1_ref, na_ref, nb_ref):
    deg = jnp.maximum(d0_ref[...] + d1_ref[...], 1.0)
    agg = jnp.concatenate([a0_ref[...], a1_ref[...]], axis=1) / deg
    h1 = jnp.maximum(hs_ref[...] + agg, 0.0)
    h1_ref[...] = h1
    hn2 = jnp.dot(h1, wn2_ref[...], preferred_element_type=jnp.float32)
    na_ref[...] = hn2[:, :H]
    nb_ref[...] = hn2[:, H:]


def _tc2(hs1b, a0, a1, d0, d1, W_neigh2):
    R = 1000
    grid = (N // R,)
    return pl.pallas_call(
        _tc2_body,
        grid=grid,
        in_specs=[
            pl.BlockSpec((R, D), lambda i: (i, 0)),
            pl.BlockSpec((R, H), lambda i: (i, 0)),
            pl.BlockSpec((R, H), lambda i: (i, 0)),
            pl.BlockSpec((R, 1), lambda i: (i, 0)),
            pl.BlockSpec((R, 1), lambda i: (i, 0)),
            pl.BlockSpec((D, D), lambda i: (0, 0)),
        ],
        out_specs=[
            pl.BlockSpec((R, D), lambda i: (i, 0)),
            pl.BlockSpec((R, H), lambda i: (i, 0)),
            pl.BlockSpec((R, H), lambda i: (i, 0)),
        ],
        out_shape=[
            jax.ShapeDtypeStruct((N, D), jnp.float32),
            jax.ShapeDtypeStruct((N, H), jnp.float32),
            jax.ShapeDtypeStruct((N, H), jnp.float32),
        ],
        compiler_params=pltpu.CompilerParams(
            dimension_semantics=("parallel",)),
    )(hs1b, a0, a1, d0, d1, W_neigh2)


def _tc3_body(h1s_ref, pa0_ref, pa1_ref, pb0_ref, pb1_ref, d0_ref, d1_ref,
              ws2_ref, b2_ref, wm1_ref, bm1_ref, g_ref, bln_ref, wm2_ref,
              bm2_ref, out_ref):
    deg = jnp.maximum(d0_ref[...] + d1_ref[...], 1.0)
    agg = jnp.concatenate(
        [pa0_ref[...] + pa1_ref[...], pb0_ref[...] + pb1_ref[...]], axis=1) / deg
    h2 = jnp.maximum(
        jnp.dot(h1s_ref[...], ws2_ref[...], preferred_element_type=jnp.float32)
        + b2_ref[...] + agg, 0.0)
    z = (jnp.dot(h2, wm1_ref[...], preferred_element_type=jnp.float32)
         + bm1_ref[...])
    mu = jnp.mean(z, axis=1, keepdims=True)
    var = jnp.mean((z - mu) * (z - mu), axis=1, keepdims=True)
    z = (z - mu) * lax.rsqrt(var + 1e-5) * g_ref[...] + bln_ref[...]
    z = jnp.maximum(z, 0.0)
    out_ref[...] = (jnp.dot(z, wm2_ref[...], preferred_element_type=jnp.float32)
                    + bm2_ref[...])


def _tc3(h1s, pa0, pa1, pb0, pb1, d0_s, d1_s, W_self2, b2, W_mlp1, b_mlp1,
         ln_g, ln_b, W_mlp2, b_mlp2):
    return pl.pallas_call(
        _tc3_body,
        out_shape=jax.ShapeDtypeStruct((NSEED, 1), jnp.float32),
    )(h1s, pa0, pa1, pb0, pb1, d0_s, d1_s, W_self2, b2, W_mlp1, b_mlp1, ln_g,
      ln_b, W_mlp2, b_mlp2)


@jax.jit
def kernel(x, node_time, seed_time, W_enc, b_enc, W_time, b_time, W_self1,
           W_neigh1, b1, W_self2, W_neigh2, b2, W_mlp1, b_mlp1, ln_g, ln_b,
           W_mlp2, b_mlp2, batch_ids, edge_index):
    src = edge_index[0].astype(jnp.int32)
    dst = edge_index[1].astype(jnp.int32)

    b01 = (b_enc + b_time).reshape(1, D)
    t0, t1, hs1b = _tc1(x, batch_ids.astype(jnp.int32).reshape(N, 1),
                        node_time.reshape(N, 1), seed_time.reshape(NSEED, 1),
                        W_enc, W_time, W_neigh1, W_self1,
                        b01, b1.reshape(1, D))

    src_p = jnp.concatenate([src, jnp.zeros((EPAD - E,), jnp.int32)])
    dst_p = jnp.concatenate([dst, jnp.full((EPAD - E,), N, jnp.int32)])
    zrow = jnp.zeros((ZROW, H), jnp.float32)
    onesr = jnp.ones((CE, H), jnp.float32)

    degp = _deg_kernel(dst_p, zrow, onesr)
    (agg,) = _agg1_kernel(src_p, dst_p, t0, t1, zrow)

    h1, na, nb = _tc2(hs1b, agg[0, :N], agg[1, :N],
                      degp[0, :N, 0:1], degp[1, :N, 0:1], W_neigh2)

    zrowb = jnp.zeros((ZROWB, H), jnp.float32)
    pa, pb = _agg2_kernel(src_p, dst_p, na, nb, zrowb)

    out = _tc3(h1[:NSEED], pa[0], pa[1], pb[0], pb[1],
               degp[0, :NSEED, 0:1], degp[1, :NSEED, 0:1],
               W_self2, b2.reshape(1, D), W_mlp1, b_mlp1.reshape(1, H),
               ln_g.reshape(1, H), ln_b.reshape(1, H), W_mlp2,
               b_mlp2.reshape(1, 1))
    return out.reshape(NSEED)
